# baseline (speedup 1.0000x reference)
import functools

import jax
import jax.numpy as jnp
from jax.experimental import pallas as pl
from jax.experimental.pallas import tpu as pltpu


def _body(rw_ref, f1b_ref, f2b_ref, outw_ref, outb_ref,
          x_any, fc1_any, fc2_any, out_ref, aux_ref,
          h_scr, x_scr, f1_scr, f2_scr, p0_scr, f2bc_scr, topi_smem,
          xsems, f1sem, f2sems, *, num_nt, tn, nb, nx):
    b = pl.program_id(0)
    s = pl.program_id(1)
    _, n, d = x_any.shape
    e_num = rw_ref.shape[0]
    xc = n // nx

    @pl.when((b == 0) & (s == 0))
    def _():
        for c in range(nx):
            pltpu.make_async_copy(
                x_any.at[0, pl.ds(c * xc, xc), :],
                x_scr.at[pl.ds(c * xc, xc), :],
                xsems.at[c],
            ).start()

    @pl.when((b == 0) & (s == 1))
    def _():
        for bb in range(1, nb):
            for c in range(nx):
                pltpu.make_async_copy(
                    x_any.at[bb, pl.ds(c * xc, xc), :],
                    x_scr.at[pl.ds((bb * nx + c) * xc, xc), :],
                    xsems.at[bb * nx + c],
                ).start()

    @pl.when(s == 0)
    def _():
        acc = None
        acc2 = None
        for c in range(nx):
            pltpu.make_async_copy(
                x_any.at[b, pl.ds(c * xc, xc), :],
                x_scr.at[pl.ds((b * nx + c) * xc, xc), :],
                xsems.at[b * nx + c],
            ).wait()
            xb = x_scr[pl.ds((b * nx + c) * xc, xc), :]
            ps = jnp.sum(xb, axis=0, keepdims=True)
            ps2 = jnp.sum(xb * xb, axis=0, keepdims=True)
            acc = ps if acc is None else acc + ps
            acc2 = ps2 if acc2 is None else acc2 + ps2
        mu = acc * (1.0 / n)
        var = acc2 * (1.0 / n) - mu * mu
        rsig = 1.0 / jnp.sqrt(var + 1e-5)
        h_scr[0:1, :] = mu
        h_scr[1:2, :] = rsig
        logits = jax.lax.dot_general(
            mu, rw_ref[...], (((1,), (1,)), ((), ())),
            preferred_element_type=jnp.float32)
        lmax = jnp.max(logits, axis=-1, keepdims=True)
        ex = jnp.exp(logits - lmax)
        probs = ex / jnp.sum(ex, axis=-1, keepdims=True)
        ii = jax.lax.broadcasted_iota(jnp.int32, (1, e_num), 1)
        pmax = jnp.max(probs, axis=-1, keepdims=True)
        top1 = jnp.min(jnp.where(probs == pmax, ii, e_num), axis=-1,
                       keepdims=True)
        e_val = top1[0, 0]
        topi_smem[b] = e_val
        pltpu.make_async_copy(fc1_any.at[e_val], f1_scr, f1sem).start()
        for c in range(num_nt):
            pltpu.make_async_copy(
                fc2_any.at[e_val, pl.ds(c * tn, tn), :],
                f2_scr.at[pl.ds(c * tn, tn), :],
                f2sems.at[c],
            ).start()

        @pl.when(b == 0)
        def _():
            p0_scr[...] = probs

        @pl.when(b == 1)
        def _():
            p0 = p0_scr[...]
            t0 = topi_smem[0]
            pm = (p0 + probs) * 0.5
            em = ((ii == t0).astype(jnp.float32)
                  + (ii == e_val).astype(jnp.float32)) * 0.5
            aux_ref[...] = e_num * jnp.sum(pm * em, axis=(0, 1),
                                           keepdims=True)

    @pl.when(s == 1)
    def _():
        e_val = topi_smem[b]
        mu = h_scr[0:1, :]
        rsig = h_scr[1:2, :]
        x_scr[pl.ds(b * n, n), :] = (x_scr[pl.ds(b * n, n), :] - mu) * rsig
        pltpu.make_async_copy(fc1_any.at[e_val], f1_scr, f1sem).wait()
        g = jax.lax.dot_general(
            f1_scr[...], x_scr[pl.ds(b * n, n), :], (((1,), (0,)), ((), ())),
            preferred_element_type=jnp.float32)
        f1b = jnp.transpose(f1b_ref[pl.ds(e_val, 1), :], (1, 0))
        f2bc_scr[...] = jnp.transpose(f2b_ref[pl.ds(e_val, 1), :], (1, 0))
        h_scr[...] = jax.nn.gelu(g + f1b, approximate=True)

    @pl.when(s >= 2)
    def _():
        e_val = topi_smem[b]
        nt = s - 2
        pltpu.make_async_copy(
            fc2_any.at[e_val, pl.ds(nt * tn, tn), :],
            f2_scr.at[pl.ds(nt * tn, tn), :],
            f2sems.at[nt],
        ).wait()
        f2t = f2_scr[pl.ds(nt * tn, tn), :]
        y = jax.lax.dot_general(
            f2t, h_scr[...], (((1,), (0,)), ((), ())),
            preferred_element_type=jnp.float32)
        y = y + f2bc_scr[pl.ds(nt * tn, tn), :]
        o = jax.lax.dot_general(
            y, outw_ref[...], (((1,), (1,)), ((), ())),
            preferred_element_type=jnp.float32)
        out_ref[0] = o + outb_ref[...]


@jax.jit
def kernel(x, router_W, fc1_W, fc1_b, fc2_W, fc2_b, out_W, out_b):
    B, N, D = x.shape
    E, H, _ = fc1_W.shape
    TN = 2048
    num_nt = N // TN
    NX = 4

    out, aux = pl.pallas_call(
        functools.partial(_body, num_nt=num_nt, tn=TN, nb=B, nx=NX),
        grid=(B, num_nt + 2),
        in_specs=[
            pl.BlockSpec((E, D), lambda b, s: (0, 0)),
            pl.BlockSpec((E, H), lambda b, s: (0, 0)),
            pl.BlockSpec((E, N), lambda b, s: (0, 0)),
            pl.BlockSpec((D, D), lambda b, s: (0, 0)),
            pl.BlockSpec((D,), lambda b, s: (0,)),
            pl.BlockSpec(memory_space=pl.ANY),
            pl.BlockSpec(memory_space=pl.ANY),
            pl.BlockSpec(memory_space=pl.ANY),
        ],
        out_specs=(
            pl.BlockSpec((1, TN, D),
                         lambda b, s: (b, jnp.maximum(s - 2, 0), 0)),
            pl.BlockSpec((1, 1), lambda b, s: (0, 0)),
        ),
        out_shape=(
            jax.ShapeDtypeStruct((B, N, D), jnp.float32),
            jax.ShapeDtypeStruct((1, 1), jnp.float32),
        ),
        scratch_shapes=[
            pltpu.VMEM((H, D), jnp.float32),
            pltpu.VMEM((B * N, D), jnp.float32),
            pltpu.VMEM((H, N), jnp.float32),
            pltpu.VMEM((N, H), jnp.float32),
            pltpu.VMEM((1, E), jnp.float32),
            pltpu.VMEM((N, 1), jnp.float32),
            pltpu.SMEM((2,), jnp.int32),
            pltpu.SemaphoreType.DMA((B * NX,)),
            pltpu.SemaphoreType.DMA,
            pltpu.SemaphoreType.DMA((num_nt,)),
        ],
    )(router_W, fc1_b, fc2_b, out_W, out_b, x, fc1_W, fc2_W)

    return out, aux[0, 0]


# speedup vs baseline: 1.0007x; 1.0007x over previous
import functools

import jax
import jax.numpy as jnp
from jax.experimental import pallas as pl
from jax.experimental.pallas import tpu as pltpu


def _body(rw_ref, f1b_ref, f2b_ref, outw_ref, outb_ref,
          x_any, fc1_any, fc2_any, out_ref, aux_ref,
          h_scr, x_scr, f1_scr, f2_scr, p0_scr, f2bc_scr, topi_smem,
          xsems, f1sem, f2sems, *, num_nt, tn, nb, nx):
    b = pl.program_id(0)
    s = pl.program_id(1)
    _, n, d = x_any.shape
    e_num = rw_ref.shape[0]
    xc = n // nx

    @pl.when((b == 0) & (s == 0))
    def _():
        for c in range(nx):
            pltpu.make_async_copy(
                x_any.at[0, pl.ds(c * xc, xc), :],
                x_scr.at[pl.ds(c * xc, xc), :],
                xsems.at[c],
            ).start()

    @pl.when((b == 0) & (s == 1))
    def _():
        for bb in range(1, nb):
            for c in range(nx):
                pltpu.make_async_copy(
                    x_any.at[bb, pl.ds(c * xc, xc), :],
                    x_scr.at[pl.ds((bb * nx + c) * xc, xc), :],
                    xsems.at[bb * nx + c],
                ).start()

    @pl.when(s == 0)
    def _():
        acc = None
        acc2 = None
        for c in range(nx):
            pltpu.make_async_copy(
                x_any.at[b, pl.ds(c * xc, xc), :],
                x_scr.at[pl.ds((b * nx + c) * xc, xc), :],
                xsems.at[b * nx + c],
            ).wait()
            xb = x_scr[pl.ds((b * nx + c) * xc, xc), :]
            ps = jnp.sum(xb, axis=0, keepdims=True)
            ps2 = jnp.sum(xb * xb, axis=0, keepdims=True)
            acc = ps if acc is None else acc + ps
            acc2 = ps2 if acc2 is None else acc2 + ps2
        mu = acc * (1.0 / n)
        var = acc2 * (1.0 / n) - mu * mu
        rsig = 1.0 / jnp.sqrt(var + 1e-5)
        h_scr[0:1, :] = mu
        h_scr[1:2, :] = rsig
        logits = jax.lax.dot_general(
            mu, rw_ref[...], (((1,), (1,)), ((), ())),
            preferred_element_type=jnp.float32)
        lmax = jnp.max(logits, axis=-1, keepdims=True)
        ex = jnp.exp(logits - lmax)
        probs = ex / jnp.sum(ex, axis=-1, keepdims=True)
        ii = jax.lax.broadcasted_iota(jnp.int32, (1, e_num), 1)
        pmax = jnp.max(probs, axis=-1, keepdims=True)
        top1 = jnp.min(jnp.where(probs == pmax, ii, e_num), axis=-1,
                       keepdims=True)
        e_val = top1[0, 0]
        topi_smem[b] = e_val
        pltpu.make_async_copy(fc1_any.at[e_val], f1_scr, f1sem).start()
        for c in range(num_nt):
            pltpu.make_async_copy(
                fc2_any.at[e_val, pl.ds(c * tn, tn), :],
                f2_scr.at[pl.ds(c * tn, tn), :],
                f2sems.at[c],
            ).start()

        @pl.when(b == 0)
        def _():
            p0_scr[...] = probs

        @pl.when(b == 1)
        def _():
            p0 = p0_scr[...]
            t0 = topi_smem[0]
            pm = (p0 + probs) * 0.5
            em = ((ii == t0).astype(jnp.float32)
                  + (ii == e_val).astype(jnp.float32)) * 0.5
            aux_ref[...] = e_num * jnp.sum(pm * em, axis=(0, 1),
                                           keepdims=True)

    @pl.when(s == 1)
    def _():
        e_val = topi_smem[b]
        mu = h_scr[0:1, :]
        rsig = h_scr[1:2, :]
        x_scr[pl.ds(b * n, n), :] = (x_scr[pl.ds(b * n, n), :] - mu) * rsig
        pltpu.make_async_copy(fc1_any.at[e_val], f1_scr, f1sem).wait()
        g = jax.lax.dot_general(
            f1_scr[...], x_scr[pl.ds(b * n, n), :], (((1,), (0,)), ((), ())),
            preferred_element_type=jnp.float32)
        f1b = jnp.transpose(f1b_ref[pl.ds(e_val, 1), :], (1, 0))
        f2bc_scr[...] = jnp.transpose(f2b_ref[pl.ds(e_val, 1), :], (1, 0))
        h_scr[...] = jax.nn.gelu(g + f1b, approximate=True)

    @pl.when(s >= 2)
    def _():
        e_val = topi_smem[b]
        nt = s - 2
        pltpu.make_async_copy(
            fc2_any.at[e_val, pl.ds(nt * tn, tn), :],
            f2_scr.at[pl.ds(nt * tn, tn), :],
            f2sems.at[nt],
        ).wait()
        f2t = f2_scr[pl.ds(nt * tn, tn), :]
        y = jax.lax.dot_general(
            f2t, h_scr[...], (((1,), (0,)), ((), ())),
            preferred_element_type=jnp.float32)
        y = y + f2bc_scr[pl.ds(nt * tn, tn), :]
        o = jax.lax.dot_general(
            y, outw_ref[...], (((1,), (1,)), ((), ())),
            preferred_element_type=jnp.float32)
        out_ref[0] = o + outb_ref[...]


@jax.jit
def kernel(x, router_W, fc1_W, fc1_b, fc2_W, fc2_b, out_W, out_b):
    B, N, D = x.shape
    E, H, _ = fc1_W.shape
    TN = 1024
    num_nt = N // TN
    NX = 8

    out, aux = pl.pallas_call(
        functools.partial(_body, num_nt=num_nt, tn=TN, nb=B, nx=NX),
        grid=(B, num_nt + 2),
        in_specs=[
            pl.BlockSpec((E, D), lambda b, s: (0, 0)),
            pl.BlockSpec((E, H), lambda b, s: (0, 0)),
            pl.BlockSpec((E, N), lambda b, s: (0, 0)),
            pl.BlockSpec((D, D), lambda b, s: (0, 0)),
            pl.BlockSpec((D,), lambda b, s: (0,)),
            pl.BlockSpec(memory_space=pl.ANY),
            pl.BlockSpec(memory_space=pl.ANY),
            pl.BlockSpec(memory_space=pl.ANY),
        ],
        out_specs=(
            pl.BlockSpec((1, TN, D),
                         lambda b, s: (b, jnp.maximum(s - 2, 0), 0)),
            pl.BlockSpec((1, 1), lambda b, s: (0, 0)),
        ),
        out_shape=(
            jax.ShapeDtypeStruct((B, N, D), jnp.float32),
            jax.ShapeDtypeStruct((1, 1), jnp.float32),
        ),
        scratch_shapes=[
            pltpu.VMEM((H, D), jnp.float32),
            pltpu.VMEM((B * N, D), jnp.float32),
            pltpu.VMEM((H, N), jnp.float32),
            pltpu.VMEM((N, H), jnp.float32),
            pltpu.VMEM((1, E), jnp.float32),
            pltpu.VMEM((N, 1), jnp.float32),
            pltpu.SMEM((2,), jnp.int32),
            pltpu.SemaphoreType.DMA((B * NX,)),
            pltpu.SemaphoreType.DMA,
            pltpu.SemaphoreType.DMA((num_nt,)),
        ],
    )(router_W, fc1_b, fc2_b, out_W, out_b, x, fc1_W, fc2_W)

    return out, aux[0, 0]


# speedup vs baseline: 1.0136x; 1.0128x over previous
import functools

import jax
import jax.numpy as jnp
from jax.experimental import pallas as pl
from jax.experimental.pallas import tpu as pltpu


def _body(rw_ref, f1b_ref, f2b_ref, outw_ref, outb_ref,
          x_any, fc1_any, fc2_any, out_ref, aux_ref,
          h_scr, x_scr, f1_scr, f2_scr, p0_scr, f2bc_scr, topi_smem,
          xsems, f1sem, f2sems, *, num_nt, tn, nb, nx):
    b = pl.program_id(0)
    s = pl.program_id(1)
    _, n, d = x_any.shape
    e_num = rw_ref.shape[0]
    xc = n // nx

    @pl.when((b == 0) & (s == 0))
    def _():
        for c in range(nx):
            pltpu.make_async_copy(
                x_any.at[0, pl.ds(c * xc, xc), :],
                x_scr.at[pl.ds(c * xc, xc), :],
                xsems.at[c],
            ).start()

    @pl.when((b == 0) & (s == 1))
    def _():
        for bb in range(1, nb):
            for c in range(nx):
                pltpu.make_async_copy(
                    x_any.at[bb, pl.ds(c * xc, xc), :],
                    x_scr.at[pl.ds((bb * nx + c) * xc, xc), :],
                    xsems.at[bb * nx + c],
                ).start()

    @pl.when(s == 0)
    def _():
        acc = None
        acc2 = None
        for c in range(nx):
            pltpu.make_async_copy(
                x_any.at[b, pl.ds(c * xc, xc), :],
                x_scr.at[pl.ds((b * nx + c) * xc, xc), :],
                xsems.at[b * nx + c],
            ).wait()
            xb = x_scr[pl.ds((b * nx + c) * xc, xc), :]
            ps = jnp.sum(xb, axis=0, keepdims=True)
            ps2 = jnp.sum(xb * xb, axis=0, keepdims=True)
            acc = ps if acc is None else acc + ps
            acc2 = ps2 if acc2 is None else acc2 + ps2
        mu = acc * (1.0 / n)
        var = acc2 * (1.0 / n) - mu * mu
        rsig = 1.0 / jnp.sqrt(var + 1e-5)
        h_scr[0:1, :] = mu
        h_scr[1:2, :] = rsig
        logits = jax.lax.dot_general(
            mu, rw_ref[...], (((1,), (1,)), ((), ())),
            preferred_element_type=jnp.float32)
        lmax = jnp.max(logits, axis=-1, keepdims=True)
        ex = jnp.exp(logits - lmax)
        probs = ex / jnp.sum(ex, axis=-1, keepdims=True)
        ii = jax.lax.broadcasted_iota(jnp.int32, (1, e_num), 1)
        pmax = jnp.max(probs, axis=-1, keepdims=True)
        top1 = jnp.min(jnp.where(probs == pmax, ii, e_num), axis=-1,
                       keepdims=True)
        e_val = top1[0, 0]
        topi_smem[b] = e_val
        pltpu.make_async_copy(fc1_any.at[e_val], f1_scr, f1sem).start()
        for c in range(num_nt):
            pltpu.make_async_copy(
                fc2_any.at[e_val, pl.ds(c * tn, tn), :],
                f2_scr.at[pl.ds(c * tn, tn), :],
                f2sems.at[c],
            ).start()

        @pl.when(b == 0)
        def _():
            p0_scr[...] = probs

        @pl.when(b == 1)
        def _():
            p0 = p0_scr[...]
            t0 = topi_smem[0]
            pm = (p0 + probs) * 0.5
            em = ((ii == t0).astype(jnp.float32)
                  + (ii == e_val).astype(jnp.float32)) * 0.5
            aux_ref[...] = e_num * jnp.sum(pm * em, axis=(0, 1),
                                           keepdims=True)

    @pl.when(s == 1)
    def _():
        e_val = topi_smem[b]
        mu = h_scr[0:1, :]
        rsig = h_scr[1:2, :]
        x_scr[pl.ds(b * n, n), :] = (x_scr[pl.ds(b * n, n), :] - mu) * rsig
        pltpu.make_async_copy(fc1_any.at[e_val], f1_scr, f1sem).wait()
        g = jax.lax.dot_general(
            f1_scr[...], x_scr[pl.ds(b * n, n), :], (((1,), (0,)), ((), ())),
            preferred_element_type=jnp.float32)
        f1b = jnp.transpose(f1b_ref[pl.ds(e_val, 1), :], (1, 0))
        f2bc_scr[...] = jnp.transpose(f2b_ref[pl.ds(e_val, 1), :], (1, 0))
        h_scr[...] = jax.nn.gelu(g + f1b, approximate=True)

    @pl.when(s >= 2)
    def _():
        e_val = topi_smem[b]
        nt = s - 2
        pltpu.make_async_copy(
            fc2_any.at[e_val, pl.ds(nt * tn, tn), :],
            f2_scr.at[pl.ds(nt * tn, tn), :],
            f2sems.at[nt],
        ).wait()
        f2t = f2_scr[pl.ds(nt * tn, tn), :]
        y = jax.lax.dot_general(
            f2t, h_scr[...], (((1,), (0,)), ((), ())),
            preferred_element_type=jnp.float32)
        y = y + f2bc_scr[pl.ds(nt * tn, tn), :]
        o = jax.lax.dot_general(
            y, outw_ref[...], (((1,), (1,)), ((), ())),
            preferred_element_type=jnp.float32)
        out_ref[0] = o + outb_ref[...]


@jax.jit
def kernel(x, router_W, fc1_W, fc1_b, fc2_W, fc2_b, out_W, out_b):
    B, N, D = x.shape
    E, H, _ = fc1_W.shape
    TN = 1024
    num_nt = N // TN
    NX = 2

    out, aux = pl.pallas_call(
        functools.partial(_body, num_nt=num_nt, tn=TN, nb=B, nx=NX),
        grid=(B, num_nt + 2),
        in_specs=[
            pl.BlockSpec((E, D), lambda b, s: (0, 0)),
            pl.BlockSpec((E, H), lambda b, s: (0, 0)),
            pl.BlockSpec((E, N), lambda b, s: (0, 0)),
            pl.BlockSpec((D, D), lambda b, s: (0, 0)),
            pl.BlockSpec((D,), lambda b, s: (0,)),
            pl.BlockSpec(memory_space=pl.ANY),
            pl.BlockSpec(memory_space=pl.ANY),
            pl.BlockSpec(memory_space=pl.ANY),
        ],
        out_specs=(
            pl.BlockSpec((1, TN, D),
                         lambda b, s: (b, jnp.maximum(s - 2, 0), 0)),
            pl.BlockSpec((1, 1), lambda b, s: (0, 0)),
        ),
        out_shape=(
            jax.ShapeDtypeStruct((B, N, D), jnp.float32),
            jax.ShapeDtypeStruct((1, 1), jnp.float32),
        ),
        scratch_shapes=[
            pltpu.VMEM((H, D), jnp.float32),
            pltpu.VMEM((B * N, D), jnp.float32),
            pltpu.VMEM((H, N), jnp.float32),
            pltpu.VMEM((N, H), jnp.float32),
            pltpu.VMEM((1, E), jnp.float32),
            pltpu.VMEM((N, 1), jnp.float32),
            pltpu.SMEM((2,), jnp.int32),
            pltpu.SemaphoreType.DMA((B * NX,)),
            pltpu.SemaphoreType.DMA,
            pltpu.SemaphoreType.DMA((num_nt,)),
        ],
    )(router_W, fc1_b, fc2_b, out_W, out_b, x, fc1_W, fc2_W)

    return out, aux[0, 0]
